# baseline (speedup 1.0000x reference)
import functools

import jax
import jax.numpy as jnp
from jax import lax
from jax.experimental import pallas as pl
from jax.experimental.pallas import tpu as pltpu
from jax.experimental.pallas import tpu_sc as plsc

_N = 10000
_M = 32
_K = 8
_DD = 16
_D = _K * _DD
_RI = 6

_BN = 400
_GRID = _N // _BN

_NW = 32
_CHUNK = 80
_NSPLIT = 5200


def _seg_mat():
    r = lax.broadcasted_iota(jnp.int32, (_D, _D), 0) // _DD
    c = lax.broadcasted_iota(jnp.int32, (_D, _D), 1) // _DD
    return (r == c).astype(jnp.float32)


def _compact_mat():
    r = lax.broadcasted_iota(jnp.int32, (_D, _K), 0) // _DD
    c = lax.broadcasted_iota(jnp.int32, (_D, _K), 1)
    return (r == c).astype(jnp.float32)


def _expand_mat():
    r = lax.broadcasted_iota(jnp.int32, (_K, _D), 0)
    c = lax.broadcasted_iota(jnp.int32, (_K, _D), 1) // _DD
    return (r == c).astype(jnp.float32)


def _seg8_mat():
    mk = _M * _K
    r = lax.broadcasted_iota(jnp.int32, (mk, mk), 0) // _K
    c = lax.broadcasted_iota(jnp.int32, (mk, mk), 1) // _K
    return (r == c).astype(jnp.float32)


def _mm(a, b):
    return lax.dot_general(a, b, (((1,), (0,)), ((), ())),
                           preferred_element_type=jnp.float32)


def _gsum(v, sg):
    return lax.dot_general(v, sg, (((1,), (0,)), ((), ())),
                           preferred_element_type=jnp.float32)


def _capsnorm(v, sg):
    s = _gsum(v * v, sg)
    return v * lax.rsqrt(jnp.maximum(s, 1e-24))


def _pre_body(x_ref, o_ref):
    sg = _seg_mat()
    o_ref[...] = _capsnorm(x_ref[...], sg)


def _normalize_input(x):
    return pl.pallas_call(
        _pre_body,
        out_shape=jax.ShapeDtypeStruct((_N, _D), jnp.float32),
    )(x)


def _route_body(x_ref, z_ref, *rest, fc):
    if fc:
        w_ref, b_ref, o_ref = rest
    else:
        (o_ref,) = rest
    sg = _seg_mat()
    sc = _compact_mat()
    ex = _expand_mat()
    s8 = _seg8_mat()
    x = x_ref[...]
    z = z_ref[...]
    zh = z.astype(jnp.bfloat16)
    u = x
    for it in range(_RI):
        uh = u.astype(jnp.bfloat16)
        ue = jnp.broadcast_to(uh[:, None, :], (_BN, _M, _D)).reshape(_BN * _M, _D)
        lt = lax.dot_general(sc.astype(jnp.bfloat16), zh * ue,
                             (((0,), (1,)), ((), ())),
                             preferred_element_type=jnp.float32)
        e = jnp.exp(lt)
        den = jnp.sum(e, axis=0, keepdims=True)
        wt = e / den
        w8 = lax.dot_general(wt, ex, (((0,), (0,)), ((), ())),
                             preferred_element_type=jnp.float32)
        s = (z * w8).reshape(_BN, _M, _D)
        s = s[:, :16] + s[:, 16:]
        s = s[:, :8] + s[:, 8:]
        u = jnp.sum(s, axis=1) + x
        if it < _RI - 1:
            u = _capsnorm(u, sg)
    h = jnp.maximum(u, 0.0)
    if fc:
        y = lax.dot_general(h, w_ref[...], (((1,), (0,)), ((), ())),
                            preferred_element_type=jnp.float32)
        y = jnp.maximum(y + b_ref[...], 0.0)
        h = _capsnorm(y, sg)
    o_ref[...] = h


def _route(xn, z, w=None, b=None, base=0, nodes=_N):
    fc = w is not None
    bb = base // _BN
    in_specs = [
        pl.BlockSpec((_BN, _D), lambda i: (i + bb, 0)),
        pl.BlockSpec((_BN * _M, _D), lambda i: (i, 0)),
    ]
    args = [xn, z]
    if fc:
        in_specs += [
            pl.BlockSpec((_D, _D), lambda i: (0, 0)),
            pl.BlockSpec((1, _D), lambda i: (0, 0)),
        ]
        args += [w, b.reshape(1, _D)]
    return pl.pallas_call(
        functools.partial(_route_body, fc=fc),
        grid=(nodes // _BN,),
        in_specs=in_specs,
        out_specs=pl.BlockSpec((_BN, _D), lambda i: (i, 0)),
        out_shape=jax.ShapeDtypeStruct((nodes, _D), jnp.float32),
    )(*args)


def _gather_body(table, idx_hbm, out, idx_v, buf0, buf1, sem0, sem1, *, nch):
    wid = lax.axis_index("s") * 2 + lax.axis_index("c")
    rpw = nch * _CHUNK
    base = wid * rpw
    pltpu.sync_copy(idx_hbm.at[pl.ds(base, rpw)], idx_v)

    def start(c, buf, sem):
        pltpu.make_async_copy(
            table.at[idx_v.at[pl.ds(c * _CHUNK, _CHUNK)]], buf, sem).start()

    def drain(c, buf, sem):
        pltpu.make_async_copy(
            table.at[idx_v.at[pl.ds(c * _CHUNK, _CHUNK)]], buf, sem).wait()
        pltpu.sync_copy(buf, out.at[pl.ds(base + c * _CHUNK, _CHUNK)])

    start(0, buf0, sem0)

    def body(t, carry):
        c = 2 * t

        @pl.when(c + 1 < nch)
        def _():
            start(c + 1, buf1, sem1)

        drain(c, buf0, sem0)

        @pl.when(c + 2 < nch)
        def _():
            start(c + 2, buf0, sem0)

        @pl.when(c + 1 < nch)
        def _():
            drain(c + 1, buf1, sem1)

        return carry

    lax.fori_loop(0, (nch + 1) // 2, body, 0)


def _gather(xn, nb_part):
    rows = nb_part.shape[0]
    nch = rows // (_NW * _CHUNK)
    assert nch * _NW * _CHUNK == rows
    k = functools.partial(
        pl.kernel,
        out_type=jax.ShapeDtypeStruct((rows, _D), jnp.float32),
        mesh=plsc.VectorSubcoreMesh(core_axis_name="c", subcore_axis_name="s"),
        scratch_types=[
            pltpu.VMEM((nch * _CHUNK,), jnp.int32),
            pltpu.VMEM((_CHUNK, _D), jnp.float32),
            pltpu.VMEM((_CHUNK, _D), jnp.float32),
            pltpu.SemaphoreType.DMA,
            pltpu.SemaphoreType.DMA,
        ],
    )(functools.partial(_gather_body, nch=nch))
    return k(xn, nb_part)


def _layer(xn, nb_a, nb_b, w, b):
    za = _gather(xn, nb_a)
    zb = _gather(xn, nb_b)
    ha = _route(xn, za, w, b, base=0, nodes=_NSPLIT)
    hb = _route(xn, zb, w, b, base=_NSPLIT, nodes=_N - _NSPLIT)
    return jnp.concatenate([ha, hb], axis=0)


def kernel(x, nb, fc1_w, fc1_b, fc2_w, fc2_b):
    nb_flat = nb.reshape(-1)
    nb_a = nb_flat[: _NSPLIT * _M]
    nb_b = nb_flat[_NSPLIT * _M:]
    xn = _normalize_input(x)
    xn = _layer(xn, nb_a, nb_b, fc1_w, fc1_b)
    xn = _layer(xn, nb_a, nb_b, fc2_w, fc2_b)
    return _layer(xn, nb_a, nb_b, None, None)


# speedup vs baseline: 1.3337x; 1.3337x over previous
import functools

import jax
import jax.numpy as jnp
from jax import lax
from jax.experimental import pallas as pl
from jax.experimental.pallas import tpu as pltpu
from jax.experimental.pallas import tpu_sc as plsc

_N = 10000
_M = 32
_K = 8
_DD = 16
_D = _K * _DD
_RI = 6

_BN = 400
_GRID = _N // _BN

_NW = 32
_CHUNK = 80
_NSPLIT = 3600


def _seg_mat():
    r = lax.broadcasted_iota(jnp.int32, (_D, _D), 0) // _DD
    c = lax.broadcasted_iota(jnp.int32, (_D, _D), 1) // _DD
    return (r == c).astype(jnp.float32)


def _compact_mat():
    r = lax.broadcasted_iota(jnp.int32, (_D, _K), 0) // _DD
    c = lax.broadcasted_iota(jnp.int32, (_D, _K), 1)
    return (r == c).astype(jnp.float32)


def _expand_mat():
    r = lax.broadcasted_iota(jnp.int32, (_K, _D), 0)
    c = lax.broadcasted_iota(jnp.int32, (_K, _D), 1) // _DD
    return (r == c).astype(jnp.float32)


def _seg8_mat():
    mk = _M * _K
    r = lax.broadcasted_iota(jnp.int32, (mk, mk), 0) // _K
    c = lax.broadcasted_iota(jnp.int32, (mk, mk), 1) // _K
    return (r == c).astype(jnp.float32)


def _mm(a, b):
    return lax.dot_general(a, b, (((1,), (0,)), ((), ())),
                           preferred_element_type=jnp.float32)


def _gsum(v, sg):
    return lax.dot_general(v, sg, (((1,), (0,)), ((), ())),
                           preferred_element_type=jnp.float32)


def _capsnorm(v, sg):
    s = _gsum(v * v, sg)
    return v * lax.rsqrt(jnp.maximum(s, 1e-24))


def _pre_body(x_ref, o_ref):
    sg = _seg_mat()
    o_ref[...] = _capsnorm(x_ref[...], sg)


def _normalize_input(x):
    return pl.pallas_call(
        _pre_body,
        out_shape=jax.ShapeDtypeStruct((_N, _D), jnp.float32),
    )(x)


def _route_body(x_ref, z_ref, *rest, fc):
    if fc:
        w_ref, b_ref, o_ref = rest
    else:
        (o_ref,) = rest
    sg = _seg_mat()
    sc = _compact_mat()
    ex = _expand_mat()
    s8 = _seg8_mat()
    x = x_ref[...]
    z = z_ref[...]
    u = x
    for it in range(_RI):
        ue = jnp.broadcast_to(u[:, None, :], (_BN, _M, _D)).reshape(_BN * _M, _D)
        lt = lax.dot_general(sc, z * ue, (((0,), (1,)), ((), ())),
                             preferred_element_type=jnp.float32)
        e = jnp.exp(lt)
        den = jnp.sum(e, axis=0, keepdims=True)
        wt = e / den
        w8 = lax.dot_general(wt, ex, (((0,), (0,)), ((), ())),
                             preferred_element_type=jnp.float32)
        s = (z * w8).reshape(_BN, _M, _D)
        s = s[:, :16] + s[:, 16:]
        s = s[:, :8] + s[:, 8:]
        u = jnp.sum(s, axis=1) + x
        if it < _RI - 1:
            u = _capsnorm(u, sg)
    h = jnp.maximum(u, 0.0)
    if fc:
        y = lax.dot_general(h, w_ref[...], (((1,), (0,)), ((), ())),
                            preferred_element_type=jnp.float32)
        y = jnp.maximum(y + b_ref[...], 0.0)
        h = _capsnorm(y, sg)
    o_ref[...] = h


def _route(xn, z, w=None, b=None, base=0, nodes=_N):
    fc = w is not None
    bb = base // _BN
    in_specs = [
        pl.BlockSpec((_BN, _D), lambda i: (i + bb, 0)),
        pl.BlockSpec((_BN * _M, _D), lambda i: (i, 0)),
    ]
    args = [xn, z]
    if fc:
        in_specs += [
            pl.BlockSpec((_D, _D), lambda i: (0, 0)),
            pl.BlockSpec((1, _D), lambda i: (0, 0)),
        ]
        args += [w, b.reshape(1, _D)]
    return pl.pallas_call(
        functools.partial(_route_body, fc=fc),
        grid=(nodes // _BN,),
        in_specs=in_specs,
        out_specs=pl.BlockSpec((_BN, _D), lambda i: (i, 0)),
        out_shape=jax.ShapeDtypeStruct((nodes, _D), jnp.float32),
    )(*args)


def _gather_body(table, idx_hbm, out, idx_v, buf0, buf1, sem0, sem1, *, nch):
    wid = lax.axis_index("s") * 2 + lax.axis_index("c")
    rpw = nch * _CHUNK
    base = wid * rpw
    pltpu.sync_copy(idx_hbm.at[pl.ds(base, rpw)], idx_v)

    def start(c, buf, sem):
        pltpu.make_async_copy(
            table.at[idx_v.at[pl.ds(c * _CHUNK, _CHUNK)]], buf, sem).start()

    def drain(c, buf, sem):
        pltpu.make_async_copy(
            table.at[idx_v.at[pl.ds(c * _CHUNK, _CHUNK)]], buf, sem).wait()
        pltpu.sync_copy(buf, out.at[pl.ds(base + c * _CHUNK, _CHUNK)])

    start(0, buf0, sem0)

    def body(t, carry):
        c = 2 * t

        @pl.when(c + 1 < nch)
        def _():
            start(c + 1, buf1, sem1)

        drain(c, buf0, sem0)

        @pl.when(c + 2 < nch)
        def _():
            start(c + 2, buf0, sem0)

        @pl.when(c + 1 < nch)
        def _():
            drain(c + 1, buf1, sem1)

        return carry

    lax.fori_loop(0, (nch + 1) // 2, body, 0)


def _gather(xn, nb_part):
    rows = nb_part.shape[0]
    nch = rows // (_NW * _CHUNK)
    assert nch * _NW * _CHUNK == rows
    k = functools.partial(
        pl.kernel,
        out_type=jax.ShapeDtypeStruct((rows, _D), jnp.float32),
        mesh=plsc.VectorSubcoreMesh(core_axis_name="c", subcore_axis_name="s"),
        scratch_types=[
            pltpu.VMEM((nch * _CHUNK,), jnp.int32),
            pltpu.VMEM((_CHUNK, _D), jnp.float32),
            pltpu.VMEM((_CHUNK, _D), jnp.float32),
            pltpu.SemaphoreType.DMA,
            pltpu.SemaphoreType.DMA,
        ],
    )(functools.partial(_gather_body, nch=nch))
    return k(xn, nb_part)


def _layer(xn, nb_a, nb_b, w, b):
    za = _gather(xn, nb_a)
    zb = _gather(xn, nb_b)
    ha = _route(xn, za, w, b, base=0, nodes=_NSPLIT)
    hb = _route(xn, zb, w, b, base=_NSPLIT, nodes=_N - _NSPLIT)
    return jnp.concatenate([ha, hb], axis=0)


def kernel(x, nb, fc1_w, fc1_b, fc2_w, fc2_b):
    nb_flat = nb.reshape(-1)
    nb_a = nb_flat[: _NSPLIT * _M]
    nb_b = nb_flat[_NSPLIT * _M:]
    xn = _normalize_input(x)
    xn = _layer(xn, nb_a, nb_b, fc1_w, fc1_b)
    xn = _layer(xn, nb_a, nb_b, fc2_w, fc2_b)
    return _layer(xn, nb_a, nb_b, None, None)


# speedup vs baseline: 1.3647x; 1.0232x over previous
import functools

import jax
import jax.numpy as jnp
from jax import lax
from jax.experimental import pallas as pl
from jax.experimental.pallas import tpu as pltpu
from jax.experimental.pallas import tpu_sc as plsc

_N = 10000
_M = 32
_K = 8
_DD = 16
_D = _K * _DD
_RI = 6

_BN = 400
_GRID = _N // _BN

_NW = 32
_CHUNK = 80
_NSPLIT = 4400


def _seg_mat():
    r = lax.broadcasted_iota(jnp.int32, (_D, _D), 0) // _DD
    c = lax.broadcasted_iota(jnp.int32, (_D, _D), 1) // _DD
    return (r == c).astype(jnp.float32)


def _compact_mat():
    r = lax.broadcasted_iota(jnp.int32, (_D, _K), 0) // _DD
    c = lax.broadcasted_iota(jnp.int32, (_D, _K), 1)
    return (r == c).astype(jnp.float32)


def _expand_mat():
    r = lax.broadcasted_iota(jnp.int32, (_K, _D), 0)
    c = lax.broadcasted_iota(jnp.int32, (_K, _D), 1) // _DD
    return (r == c).astype(jnp.float32)


def _seg8_mat():
    mk = _M * _K
    r = lax.broadcasted_iota(jnp.int32, (mk, mk), 0) // _K
    c = lax.broadcasted_iota(jnp.int32, (mk, mk), 1) // _K
    return (r == c).astype(jnp.float32)


def _mm(a, b):
    return lax.dot_general(a, b, (((1,), (0,)), ((), ())),
                           preferred_element_type=jnp.float32)


def _gsum(v, sg):
    return lax.dot_general(v, sg, (((1,), (0,)), ((), ())),
                           preferred_element_type=jnp.float32)


def _capsnorm(v, sg):
    s = _gsum(v * v, sg)
    return v * lax.rsqrt(jnp.maximum(s, 1e-24))


def _pre_body(x_ref, o_ref):
    sg = _seg_mat()
    o_ref[...] = _capsnorm(x_ref[...], sg)


def _normalize_input(x):
    return pl.pallas_call(
        _pre_body,
        out_shape=jax.ShapeDtypeStruct((_N, _D), jnp.float32),
    )(x)


def _route_body(x_ref, z_ref, *rest, fc):
    if fc:
        w_ref, b_ref, o_ref = rest
    else:
        (o_ref,) = rest
    sg = _seg_mat()
    sc = _compact_mat()
    ex = _expand_mat()
    s8 = _seg8_mat()
    x = x_ref[...]
    z = z_ref[...]
    u = x
    for it in range(_RI):
        ue = jnp.broadcast_to(u[:, None, :], (_BN, _M, _D)).reshape(_BN * _M, _D)
        lt = lax.dot_general(sc, z * ue, (((0,), (1,)), ((), ())),
                             preferred_element_type=jnp.float32)
        e = jnp.exp(lt)
        den = jnp.sum(e, axis=0, keepdims=True)
        wt = e / den
        w8 = lax.dot_general(wt, ex, (((0,), (0,)), ((), ())),
                             preferred_element_type=jnp.float32)
        s = (z * w8).reshape(_BN, _M, _D)
        s = s[:, :16] + s[:, 16:]
        s = s[:, :8] + s[:, 8:]
        u = jnp.sum(s, axis=1) + x
        if it < _RI - 1:
            u = _capsnorm(u, sg)
    h = jnp.maximum(u, 0.0)
    if fc:
        y = lax.dot_general(h, w_ref[...], (((1,), (0,)), ((), ())),
                            preferred_element_type=jnp.float32)
        y = jnp.maximum(y + b_ref[...], 0.0)
        h = _capsnorm(y, sg)
    o_ref[...] = h


def _route(xn, z, w=None, b=None, base=0, nodes=_N):
    fc = w is not None
    bb = base // _BN
    in_specs = [
        pl.BlockSpec((_BN, _D), lambda i: (i + bb, 0)),
        pl.BlockSpec((_BN * _M, _D), lambda i: (i, 0)),
    ]
    args = [xn, z]
    if fc:
        in_specs += [
            pl.BlockSpec((_D, _D), lambda i: (0, 0)),
            pl.BlockSpec((1, _D), lambda i: (0, 0)),
        ]
        args += [w, b.reshape(1, _D)]
    return pl.pallas_call(
        functools.partial(_route_body, fc=fc),
        grid=(nodes // _BN,),
        in_specs=in_specs,
        out_specs=pl.BlockSpec((_BN, _D), lambda i: (i, 0)),
        out_shape=jax.ShapeDtypeStruct((nodes, _D), jnp.float32),
    )(*args)


def _gather_body(table, idx_hbm, out, idx_v, buf0, buf1, sem0, sem1, *, nch):
    wid = lax.axis_index("s") * 2 + lax.axis_index("c")
    rpw = nch * _CHUNK
    base = wid * rpw
    pltpu.sync_copy(idx_hbm.at[pl.ds(base, rpw)], idx_v)

    def start(c, buf, sem):
        pltpu.make_async_copy(
            table.at[idx_v.at[pl.ds(c * _CHUNK, _CHUNK)]], buf, sem).start()

    def drain(c, buf, sem):
        pltpu.make_async_copy(
            table.at[idx_v.at[pl.ds(c * _CHUNK, _CHUNK)]], buf, sem).wait()
        pltpu.sync_copy(buf, out.at[pl.ds(base + c * _CHUNK, _CHUNK)])

    start(0, buf0, sem0)

    def body(t, carry):
        c = 2 * t

        @pl.when(c + 1 < nch)
        def _():
            start(c + 1, buf1, sem1)

        drain(c, buf0, sem0)

        @pl.when(c + 2 < nch)
        def _():
            start(c + 2, buf0, sem0)

        @pl.when(c + 1 < nch)
        def _():
            drain(c + 1, buf1, sem1)

        return carry

    lax.fori_loop(0, (nch + 1) // 2, body, 0)


def _gather(xn, nb_part):
    rows = nb_part.shape[0]
    nch = rows // (_NW * _CHUNK)
    assert nch * _NW * _CHUNK == rows
    k = functools.partial(
        pl.kernel,
        out_type=jax.ShapeDtypeStruct((rows, _D), jnp.float32),
        mesh=plsc.VectorSubcoreMesh(core_axis_name="c", subcore_axis_name="s"),
        scratch_types=[
            pltpu.VMEM((nch * _CHUNK,), jnp.int32),
            pltpu.VMEM((_CHUNK, _D), jnp.float32),
            pltpu.VMEM((_CHUNK, _D), jnp.float32),
            pltpu.SemaphoreType.DMA,
            pltpu.SemaphoreType.DMA,
        ],
    )(functools.partial(_gather_body, nch=nch))
    return k(xn, nb_part)


def _layer(xn, nb_a, nb_b, w, b):
    za = _gather(xn, nb_a)
    zb = _gather(xn, nb_b)
    ha = _route(xn, za, w, b, base=0, nodes=_NSPLIT)
    hb = _route(xn, zb, w, b, base=_NSPLIT, nodes=_N - _NSPLIT)
    return jnp.concatenate([ha, hb], axis=0)


def kernel(x, nb, fc1_w, fc1_b, fc2_w, fc2_b):
    nb_flat = nb.reshape(-1)
    nb_a = nb_flat[: _NSPLIT * _M]
    nb_b = nb_flat[_NSPLIT * _M:]
    xn = _normalize_input(x)
    xn = _layer(xn, nb_a, nb_b, fc1_w, fc1_b)
    xn = _layer(xn, nb_a, nb_b, fc2_w, fc2_b)
    return _layer(xn, nb_a, nb_b, None, None)


# speedup vs baseline: 1.3872x; 1.0165x over previous
import functools

import jax
import jax.numpy as jnp
from jax import lax
from jax.experimental import pallas as pl
from jax.experimental.pallas import tpu as pltpu
from jax.experimental.pallas import tpu_sc as plsc

_N = 10000
_M = 32
_K = 8
_DD = 16
_D = _K * _DD
_RI = 6

_BN = 400
_GRID = _N // _BN

_NW = 32
_CHUNK = 80
_NSPLIT = 5200


def _seg_mat():
    r = lax.broadcasted_iota(jnp.int32, (_D, _D), 0) // _DD
    c = lax.broadcasted_iota(jnp.int32, (_D, _D), 1) // _DD
    return (r == c).astype(jnp.float32)


def _compact_mat():
    r = lax.broadcasted_iota(jnp.int32, (_D, _K), 0) // _DD
    c = lax.broadcasted_iota(jnp.int32, (_D, _K), 1)
    return (r == c).astype(jnp.float32)


def _expand_mat():
    r = lax.broadcasted_iota(jnp.int32, (_K, _D), 0)
    c = lax.broadcasted_iota(jnp.int32, (_K, _D), 1) // _DD
    return (r == c).astype(jnp.float32)


def _gsum(v, sg):
    return lax.dot_general(v, sg, (((1,), (0,)), ((), ())),
                           preferred_element_type=jnp.float32)


def _capsnorm(v, sg):
    s = _gsum(v * v, sg)
    return v * lax.rsqrt(jnp.maximum(s, 1e-24))


def _pre_body(x_ref, o_ref):
    sg = _seg_mat()
    o_ref[...] = _capsnorm(x_ref[...], sg)


def _normalize_input(x):
    return pl.pallas_call(
        _pre_body,
        out_shape=jax.ShapeDtypeStruct((_N, _D), jnp.float32),
    )(x)


def _route_body(x_ref, z_ref, *rest, fc):
    if fc:
        w_ref, b_ref, o_ref = rest
    else:
        (o_ref,) = rest
    sg = _seg_mat()
    sc = _compact_mat()
    ex = _expand_mat()
    x = x_ref[...]
    z = z_ref[...]
    u = x
    for it in range(_RI):
        ue = jnp.broadcast_to(u[:, None, :], (_BN, _M, _D)).reshape(_BN * _M, _D)
        lt = lax.dot_general(sc, z * ue, (((0,), (1,)), ((), ())),
                             preferred_element_type=jnp.float32)
        e = jnp.exp(lt)
        den = jnp.sum(e, axis=0, keepdims=True)
        wt = e / den
        w8 = lax.dot_general(wt, ex, (((0,), (0,)), ((), ())),
                             preferred_element_type=jnp.float32)
        s = (z * w8).reshape(_BN, _M, _D)
        s = s[:, :16] + s[:, 16:]
        s = s[:, :8] + s[:, 8:]
        u = jnp.sum(s, axis=1) + x
        if it < _RI - 1:
            u = _capsnorm(u, sg)
    h = jnp.maximum(u, 0.0)
    if fc:
        y = lax.dot_general(h, w_ref[...], (((1,), (0,)), ((), ())),
                            preferred_element_type=jnp.float32)
        y = jnp.maximum(y + b_ref[...], 0.0)
        h = _capsnorm(y, sg)
    o_ref[...] = h


def _route(xn, z, w=None, b=None, base=0, nodes=_N):
    fc = w is not None
    bb = base // _BN
    in_specs = [
        pl.BlockSpec((_BN, _D), lambda i: (i + bb, 0)),
        pl.BlockSpec((_BN * _M, _D), lambda i: (i, 0)),
    ]
    args = [xn, z]
    if fc:
        in_specs += [
            pl.BlockSpec((_D, _D), lambda i: (0, 0)),
            pl.BlockSpec((1, _D), lambda i: (0, 0)),
        ]
        args += [w, b.reshape(1, _D)]
    return pl.pallas_call(
        functools.partial(_route_body, fc=fc),
        grid=(nodes // _BN,),
        in_specs=in_specs,
        out_specs=pl.BlockSpec((_BN, _D), lambda i: (i, 0)),
        out_shape=jax.ShapeDtypeStruct((nodes, _D), jnp.float32),
    )(*args)


def _gather_body(table, idx_hbm, out, idx_v, buf0, buf1, sem0, sem1, *, nch):
    wid = lax.axis_index("s") * 2 + lax.axis_index("c")
    rpw = nch * _CHUNK
    base = wid * rpw
    pltpu.sync_copy(idx_hbm.at[pl.ds(base, rpw)], idx_v)

    def start(c, buf, sem):
        pltpu.make_async_copy(
            table.at[idx_v.at[pl.ds(c * _CHUNK, _CHUNK)]], buf, sem).start()

    def drain(c, buf, sem):
        pltpu.make_async_copy(
            table.at[idx_v.at[pl.ds(c * _CHUNK, _CHUNK)]], buf, sem).wait()
        pltpu.sync_copy(buf, out.at[pl.ds(base + c * _CHUNK, _CHUNK)])

    start(0, buf0, sem0)

    def body(t, carry):
        c = 2 * t

        @pl.when(c + 1 < nch)
        def _():
            start(c + 1, buf1, sem1)

        drain(c, buf0, sem0)

        @pl.when(c + 2 < nch)
        def _():
            start(c + 2, buf0, sem0)

        @pl.when(c + 1 < nch)
        def _():
            drain(c + 1, buf1, sem1)

        return carry

    lax.fori_loop(0, (nch + 1) // 2, body, 0)


def _gather(xn, nb_part):
    rows = nb_part.shape[0]
    nch = rows // (_NW * _CHUNK)
    assert nch * _NW * _CHUNK == rows
    k = functools.partial(
        pl.kernel,
        out_type=jax.ShapeDtypeStruct((rows, _D), jnp.float32),
        mesh=plsc.VectorSubcoreMesh(core_axis_name="c", subcore_axis_name="s"),
        scratch_types=[
            pltpu.VMEM((nch * _CHUNK,), jnp.int32),
            pltpu.VMEM((_CHUNK, _D), jnp.float32),
            pltpu.VMEM((_CHUNK, _D), jnp.float32),
            pltpu.SemaphoreType.DMA,
            pltpu.SemaphoreType.DMA,
        ],
    )(functools.partial(_gather_body, nch=nch))
    return k(xn, nb_part)


def _layer(xn, nb_a, nb_b, w, b):
    za = _gather(xn, nb_a)
    zb = _gather(xn, nb_b)
    ha = _route(xn, za, w, b, base=0, nodes=_NSPLIT)
    hb = _route(xn, zb, w, b, base=_NSPLIT, nodes=_N - _NSPLIT)
    return jnp.concatenate([ha, hb], axis=0)


def kernel(x, nb, fc1_w, fc1_b, fc2_w, fc2_b):
    nb_flat = nb.reshape(-1)
    nb_a = nb_flat[: _NSPLIT * _M]
    nb_b = nb_flat[_NSPLIT * _M:]
    xn = _normalize_input(x)
    xn = _layer(xn, nb_a, nb_b, fc1_w, fc1_b)
    xn = _layer(xn, nb_a, nb_b, fc2_w, fc2_b)
    return _layer(xn, nb_a, nb_b, None, None)
